# baseline (speedup 1.0000x reference)
import functools

import jax
import jax.numpy as jnp
from jax import lax
from jax.experimental import pallas as pl
from jax.experimental.pallas import tpu as pltpu
from jax.experimental.pallas import tpu_sc as plsc

D = 64
W = 128
EPS = 1e-5
PAD = 0
LANES = 16
NJ = D // LANES


def _layer_norm_rows(o_vmem, i_vmem, gs, bs):
    @pl.loop(0, W // LANES)
    def _(g):
        ivs = i_vmem[0, pl.ds(LANES * g, LANES)]
        mf = jnp.where(ivs != jnp.int32(PAD), 1.0, 0.0)
        for rr in range(LANES):
            r = LANES * g + rr
            v = [o_vmem[r, pl.ds(LANES * j, LANES)] for j in range(NJ)]
            s = jnp.sum(v[0] + v[1] + v[2] + v[3])
            q = jnp.sum(v[0] * v[0] + v[1] * v[1] + v[2] * v[2] + v[3] * v[3])
            mean = s * (1.0 / D)
            var = q * (1.0 / D) - mean * mean
            t = var + EPS
            bits = lax.bitcast_convert_type(t, jnp.int32)
            bits = jnp.int32(0x5F3759DF) - lax.shift_right_logical(bits, 1)
            y = lax.bitcast_convert_type(bits, jnp.float32)
            h = 0.5 * t
            y = y * (1.5 - h * y * y)
            y = y * (1.5 - h * y * y)
            y = y * (1.5 - h * y * y)
            m = mf[rr]
            a = y * m
            b = (-mean * y) * m
            av = lax.broadcast_in_dim(a, (LANES,), ())
            bv = lax.broadcast_in_dim(b, (LANES,), ())
            for j in range(NJ):
                o_vmem[r, pl.ds(LANES * j, LANES)] = (v[j] * av + bv) * gs[j] + bs[j]


def kernel(x, table, gamma, beta):
    B, L = x.shape
    n = B * L
    idx = x.reshape(1, n)
    mesh = plsc.VectorSubcoreMesh(core_axis_name="core", subcore_axis_name="subcore")

    @functools.partial(
        pl.kernel,
        out_type=jax.ShapeDtypeStruct((n, D), table.dtype),
        mesh=mesh,
        compiler_params=pltpu.CompilerParams(
            needs_layout_passes=False, use_tc_tiling_on_sc=False
        ),
        scratch_types=[
            pltpu.VMEM((D,), jnp.float32),
            pltpu.VMEM((D,), jnp.float32),
        ],
    )
    def run(table_hbm, idx_hbm, gamma_hbm, beta_hbm, out_hbm, gamma_v, beta_v):
        pltpu.sync_copy(gamma_hbm, gamma_v)
        pltpu.sync_copy(beta_hbm, beta_v)
        gs = [gamma_v[pl.ds(LANES * j, LANES)] for j in range(NJ)]
        bs = [beta_v[pl.ds(LANES * j, LANES)] for j in range(NJ)]

        def body(i_vmem, o_vmem):
            pltpu.sync_copy(table_hbm.at[i_vmem.at[0]], o_vmem)
            _layer_norm_rows(o_vmem, i_vmem, gs, bs)

        pltpu.emit_pipeline(
            body,
            grid=(n // W,),
            in_specs=[pl.BlockSpec((1, W), lambda i: (0, i))],
            out_specs=[pl.BlockSpec((W, D), lambda i: (i, 0))],
            core_axis_name=("core", "subcore"),
            dimension_semantics=(pltpu.PARALLEL,),
        )(idx_hbm, out_hbm)

    out = run(table, idx, gamma, beta)
    return out.reshape(B, L, D)


# speedup vs baseline: 1.0507x; 1.0507x over previous
import functools

import jax
import jax.numpy as jnp
from jax import lax
from jax.experimental import pallas as pl
from jax.experimental.pallas import tpu as pltpu
from jax.experimental.pallas import tpu_sc as plsc

D = 64
W = 128
EPS = 1e-5
PAD = 0
LANES = 16
NJ = D // LANES


def _layer_norm_rows(o_vmem, i_vmem, gs, bs):
    @pl.loop(0, W // LANES)
    def _(g):
        rid = lax.iota(jnp.int32, LANES) + LANES * g
        ivs = i_vmem[0, pl.ds(LANES * g, LANES)]
        mf = jnp.where(ivs != jnp.int32(PAD), 1.0, 0.0)
        s = None
        q = None
        for c in range(D):
            col = jnp.full((LANES,), c, jnp.int32)
            xc = plsc.load_gather(o_vmem, [rid, col])
            s = xc if s is None else s + xc
            q = xc * xc if q is None else q + xc * xc
        mean = s * (1.0 / D)
        var = q * (1.0 / D) - mean * mean
        t = var + EPS
        bits = lax.bitcast_convert_type(t, jnp.int32)
        bits = jnp.int32(0x5F3759DF) - lax.shift_right_logical(bits, 1)
        y = lax.bitcast_convert_type(bits, jnp.float32)
        h = 0.5 * t
        y = y * (1.5 - h * y * y)
        y = y * (1.5 - h * y * y)
        y = y * (1.5 - h * y * y)
        a_all = y * mf
        b_all = (0.0 - mean) * y * mf
        for rr in range(LANES):
            r = LANES * g + rr
            av = lax.broadcast_in_dim(a_all[rr], (LANES,), ())
            bv = lax.broadcast_in_dim(b_all[rr], (LANES,), ())
            for j in range(NJ):
                sl = pl.ds(LANES * j, LANES)
                o_vmem[r, sl] = (o_vmem[r, sl] * av + bv) * gs[j] + bs[j]


def kernel(x, table, gamma, beta):
    B, L = x.shape
    n = B * L
    idx = x.reshape(1, n)
    mesh = plsc.VectorSubcoreMesh(core_axis_name="core", subcore_axis_name="subcore")

    @functools.partial(
        pl.kernel,
        out_type=jax.ShapeDtypeStruct((n, D), table.dtype),
        mesh=mesh,
        compiler_params=pltpu.CompilerParams(
            needs_layout_passes=False, use_tc_tiling_on_sc=False
        ),
        scratch_types=[
            pltpu.VMEM((D,), jnp.float32),
            pltpu.VMEM((D,), jnp.float32),
        ],
    )
    def run(table_hbm, idx_hbm, gamma_hbm, beta_hbm, out_hbm, gamma_v, beta_v):
        pltpu.sync_copy(gamma_hbm, gamma_v)
        pltpu.sync_copy(beta_hbm, beta_v)
        gs = [gamma_v[pl.ds(LANES * j, LANES)] for j in range(NJ)]
        bs = [beta_v[pl.ds(LANES * j, LANES)] for j in range(NJ)]

        def body(i_vmem, o_vmem):
            pltpu.sync_copy(table_hbm.at[i_vmem.at[0]], o_vmem)
            _layer_norm_rows(o_vmem, i_vmem, gs, bs)

        pltpu.emit_pipeline(
            body,
            grid=(n // W,),
            in_specs=[pl.BlockSpec((1, W), lambda i: (0, i))],
            out_specs=[pl.BlockSpec((W, D), lambda i: (i, 0))],
            core_axis_name=("core", "subcore"),
            dimension_semantics=(pltpu.PARALLEL,),
        )(idx_hbm, out_hbm)

    out = run(table, idx, gamma, beta)
    return out.reshape(B, L, D)


# speedup vs baseline: 1.1127x; 1.0590x over previous
import functools

import jax
import jax.numpy as jnp
from jax import lax
from jax.experimental import pallas as pl
from jax.experimental.pallas import tpu as pltpu
from jax.experimental.pallas import tpu_sc as plsc

D = 64
C = 256
NBUF = 4
GW = 128
EPS = 1e-5
PAD = 0
LANES = 16
NJ = D // LANES


def _layer_norm_rows(rows, idxs, gs, bs):
    @pl.loop(0, C // LANES)
    def _(g):
        rid = lax.iota(jnp.int32, LANES) + LANES * g
        ivs = idxs[pl.ds(LANES * g, LANES)]
        mf = jnp.where(ivs != jnp.int32(PAD), 1.0, 0.0)
        s = None
        q = None
        for c in range(D):
            col = jnp.full((LANES,), c, jnp.int32)
            xc = plsc.load_gather(rows, [rid, col])
            s = xc if s is None else s + xc
            q = xc * xc if q is None else q + xc * xc
        mean = s * (1.0 / D)
        var = q * (1.0 / D) - mean * mean
        t = var + EPS
        bits = lax.bitcast_convert_type(t, jnp.int32)
        bits = jnp.int32(0x5F3759DF) - lax.shift_right_logical(bits, 1)
        y = lax.bitcast_convert_type(bits, jnp.float32)
        h = 0.5 * t
        y = y * (1.5 - h * y * y)
        y = y * (1.5 - h * y * y)
        y = y * (1.5 - h * y * y)
        a_all = y * mf
        b_all = (0.0 - mean) * y * mf
        for rr in range(LANES):
            r = LANES * g + rr
            av = lax.broadcast_in_dim(a_all[rr], (LANES,), ())
            bv = lax.broadcast_in_dim(b_all[rr], (LANES,), ())
            for j in range(NJ):
                sl = pl.ds(LANES * j, LANES)
                rows[r, sl] = (rows[r, sl] * av + bv) * gs[j] + bs[j]


def kernel(x, table, gamma, beta):
    B, L = x.shape
    n = B * L
    idx = x.reshape(n)
    mesh = plsc.VectorSubcoreMesh(core_axis_name="core", subcore_axis_name="subcore")
    nworkers = 32
    nw = n // nworkers
    steps = nw // C

    row_bufs = [pltpu.VMEM((C, D), jnp.float32) for _ in range(NBUF)]
    idx_bufs = [pltpu.VMEM((C,), jnp.int32) for _ in range(NBUF)]
    gsems = [pltpu.SemaphoreType.DMA for _ in range(NBUF)]
    ssems = [pltpu.SemaphoreType.DMA for _ in range(NBUF)]

    @functools.partial(
        pl.kernel,
        out_type=jax.ShapeDtypeStruct((n, D), table.dtype),
        mesh=mesh,
        compiler_params=pltpu.CompilerParams(
            needs_layout_passes=False, use_tc_tiling_on_sc=False
        ),
        scratch_types=row_bufs
        + idx_bufs
        + gsems
        + ssems
        + [pltpu.VMEM((D,), jnp.float32), pltpu.VMEM((D,), jnp.float32)],
    )
    def run(table_hbm, idx_hbm, gamma_hbm, beta_hbm, out_hbm, *scratch):
        rbuf = scratch[:NBUF]
        ibuf = scratch[NBUF : 2 * NBUF]
        gsem = scratch[2 * NBUF : 3 * NBUF]
        ssem = scratch[3 * NBUF : 4 * NBUF]
        gamma_v, beta_v = scratch[4 * NBUF], scratch[4 * NBUF + 1]

        pltpu.sync_copy(gamma_hbm, gamma_v)
        pltpu.sync_copy(beta_hbm, beta_v)
        gs = [gamma_v[pl.ds(LANES * j, LANES)] for j in range(NJ)]
        bs = [beta_v[pl.ds(LANES * j, LANES)] for j in range(NJ)]

        wid = lax.axis_index("subcore") * 2 + lax.axis_index("core")
        base = wid * nw

        def load_and_gather(k, b):
            pltpu.sync_copy(idx_hbm.at[pl.ds(base + k * C, C)], ibuf[b])
            for j in range(C // GW):
                pltpu.async_copy(
                    table_hbm.at[ibuf[b].at[pl.ds(j * GW, GW)]],
                    rbuf[b].at[pl.ds(j * GW, GW)],
                    gsem[b],
                )

        def wait_gather(b):
            for j in range(C // GW):
                pltpu.make_async_copy(
                    table_hbm.at[ibuf[b].at[pl.ds(j * GW, GW)]],
                    rbuf[b].at[pl.ds(j * GW, GW)],
                    gsem[b],
                ).wait()

        def store(k, b):
            pltpu.async_copy(rbuf[b], out_hbm.at[pl.ds(base + k * C, C)], ssem[b])

        def wait_store(b):
            pltpu.make_async_copy(rbuf[b], out_hbm.at[pl.ds(0, C)], ssem[b]).wait()

        load_and_gather(0, 0)

        @pl.loop(0, steps, step=NBUF)
        def _(k0):
            for b in range(NBUF):
                k = k0 + b
                b_next = (b + 1) % NBUF

                @pl.when(k >= NBUF - 1)
                def _():
                    wait_store(b_next)

                @pl.when(k < steps - 1)
                def _():
                    load_and_gather(k + 1, b_next)

                wait_gather(b)
                _layer_norm_rows(rbuf[b], ibuf[b], gs, bs)
                store(k, b)

        for t in range(steps - NBUF + 1, steps):
            wait_store(t % NBUF)

    out = run(table, idx, gamma, beta)
    return out.reshape(B, L, D)


# speedup vs baseline: 2.0387x; 1.8322x over previous
import functools

import jax
import jax.numpy as jnp
from jax import lax
from jax.experimental import pallas as pl
from jax.experimental.pallas import tpu as pltpu
from jax.experimental.pallas import tpu_sc as plsc

D = 64
C = 512
NBUF = 2
GW = 128
RB = 8192
EPS = 1e-5
PAD = 0
LANES = 16
NWORKERS = 32


def _sc_gather(table, idx, n):
    mesh = plsc.VectorSubcoreMesh(core_axis_name="core", subcore_axis_name="subcore")
    nw = n // NWORKERS
    steps = nw // C

    @functools.partial(
        pl.kernel,
        out_type=jax.ShapeDtypeStruct((n, 2 * D), jnp.float32),
        mesh=mesh,
        compiler_params=pltpu.CompilerParams(
            needs_layout_passes=False, use_tc_tiling_on_sc=False
        ),
        scratch_types=[pltpu.VMEM((C, D), jnp.float32) for _ in range(NBUF)]
        + [pltpu.VMEM((C,), jnp.int32) for _ in range(NBUF)]
        + [pltpu.SemaphoreType.DMA for _ in range(2 * NBUF)],
    )
    def run(table_hbm, idx_hbm, out_hbm, *scratch):
        rbuf = scratch[:NBUF]
        ibuf = scratch[NBUF : 2 * NBUF]
        gsem = scratch[2 * NBUF : 3 * NBUF]
        ssem = scratch[3 * NBUF : 4 * NBUF]

        wid = lax.axis_index("subcore") * 2 + lax.axis_index("core")
        base = wid * nw

        def load_and_gather(k, b):
            pltpu.sync_copy(idx_hbm.at[pl.ds(base + k * C, C)], ibuf[b])
            for j in range(C // GW):
                pltpu.async_copy(
                    table_hbm.at[ibuf[b].at[pl.ds(j * GW, GW)]],
                    rbuf[b].at[pl.ds(j * GW, GW)],
                    gsem[b],
                )

        def wait_gather(b):
            for j in range(C // GW):
                pltpu.make_async_copy(
                    table_hbm.at[ibuf[b].at[pl.ds(j * GW, GW)]],
                    rbuf[b].at[pl.ds(j * GW, GW)],
                    gsem[b],
                ).wait()

        def store(k, b):
            pltpu.async_copy(
                rbuf[b],
                out_hbm.at[pl.ds(base + k * C, C), pl.ds(0, D)],
                ssem[b],
            )

        def wait_store(b):
            pltpu.make_async_copy(
                rbuf[b], out_hbm.at[pl.ds(0, C), pl.ds(0, D)], ssem[b]
            ).wait()

        def zero_padding_rows(b):
            @pl.loop(0, C // LANES)
            def _(g):
                ivs = ibuf[b][pl.ds(LANES * g, LANES)]
                haspad = jnp.any(ivs == jnp.int32(PAD))

                @pl.when(haspad)
                def _():
                    mf = jnp.where(ivs != jnp.int32(PAD), 1.0, 0.0)
                    for rr in range(LANES):
                        r = LANES * g + rr
                        bm = lax.broadcast_in_dim(mf[rr], (LANES,), ())
                        for j in range(D // LANES):
                            sl = pl.ds(LANES * j, LANES)
                            rbuf[b][r, sl] = rbuf[b][r, sl] * bm

        load_and_gather(0, 0)

        @pl.loop(0, steps, step=NBUF)
        def _(k0):
            for b in range(NBUF):
                k = k0 + b
                b_next = (b + 1) % NBUF

                @pl.when(k >= NBUF - 1)
                def _():
                    wait_store(b_next)

                @pl.when(k < steps - 1)
                def _():
                    load_and_gather(k + 1, b_next)

                wait_gather(b)
                zero_padding_rows(b)
                store(k, b)

        for t in range(steps - NBUF + 1, steps):
            wait_store(t % NBUF)

    return run(table, idx)


def _tc_layer_norm(emb, gamma, beta, n):
    def body(e_ref, g_ref, b_ref, o_ref):
        e = e_ref[...][:, :D]
        mu = jnp.mean(e, axis=-1, keepdims=True)
        d = e - mu
        var = jnp.mean(d * d, axis=-1, keepdims=True)
        y = d * lax.rsqrt(var + EPS)
        o_ref[...] = y * g_ref[...] + b_ref[...]

    return pl.pallas_call(
        body,
        grid=(n // RB,),
        in_specs=[
            pl.BlockSpec((RB, 2 * D), lambda i: (i, 0)),
            pl.BlockSpec((1, D), lambda i: (0, 0)),
            pl.BlockSpec((1, D), lambda i: (0, 0)),
        ],
        out_specs=pl.BlockSpec((RB, D), lambda i: (i, 0)),
        out_shape=jax.ShapeDtypeStruct((n, D), jnp.float32),
    )(emb, gamma.reshape(1, D), beta.reshape(1, D))


def kernel(x, table, gamma, beta):
    B, L = x.shape
    n = B * L
    emb = _sc_gather(table, x.reshape(n), n)
    out = _tc_layer_norm(emb, gamma, beta, n)
    return out.reshape(B, L, D)
